# baseline (speedup 1.0000x reference)
import jax
import jax.numpy as jnp
import numpy as np
from jax.experimental import pallas as pl

IOU_T = 0.5
CONF_T = 0.05
MAX_DET = 100
PRE_NMS = 1000
NMS_PAD = 1024
N_ANCH = 49104
N_ANCH_PAD = 49152
N_IMG = 8
BLK_A = 1024

NEG_INF = float("-inf")


def _make_anchors_np(h, w):
    aspect_ratios = [0.5, 1.0, 2.0]
    scales = [2.0 ** x for x in [0.0, 1.0 / 3.0, 2.0 / 3.0]]
    all_a = []
    for i in range(3, 8):
        area = float((2 ** (i + 2)) ** 2)
        stride = float(2 ** i)
        dims = []
        for r in aspect_ratios:
            ah = np.sqrt(area / r)
            aw = area / ah
            for s in scales:
                dims.append([aw * s, ah * s])
        dims = np.array(dims, np.float32)
        fh = int(np.ceil(h / stride))
        fw = int(np.ceil(w / stride))
        rx = (np.arange(fw, dtype=np.float32) + 0.5) * stride
        ry = (np.arange(fh, dtype=np.float32) + 0.5) * stride
        cx, cy = np.meshgrid(rx, ry)
        centers = np.stack([cx, cy], -1)
        centers = np.tile(centers[:, :, None, :], (1, 1, 9, 1))
        d = np.tile(dims[None, None, :, :], (fh, fw, 1, 1))
        all_a.append(np.concatenate([centers, d], -1).reshape(-1, 4))
    return np.concatenate(all_a, 0)


def _conf_body(pred_ref, m_ref):
    j = pl.program_id(0)
    x = pred_ref[...]
    sg = jax.nn.sigmoid(x)
    li = jax.lax.broadcasted_iota(jnp.int32, x.shape, 2)
    sgm = jnp.where(li >= 4, sg, -1.0)
    m = jnp.max(sgm, axis=-1)
    ga = j * BLK_A + jax.lax.broadcasted_iota(jnp.int32, m.shape, 1)
    m_ref[...] = jnp.where(ga < N_ANCH, m, NEG_INF)


def _confidence(predictions):
    return pl.pallas_call(
        _conf_body,
        grid=(N_ANCH_PAD // BLK_A,),
        in_specs=[pl.BlockSpec((N_IMG, BLK_A, 84), lambda j: (0, j, 0))],
        out_specs=pl.BlockSpec((N_IMG, BLK_A), lambda j: (0, j)),
        out_shape=jax.ShapeDtypeStruct((N_IMG, N_ANCH_PAD), jnp.float32),
    )(predictions)


def _nms_body(rows_ref, an_ref, s_ref, out_ref):
    rows = rows_ref[...]
    sg = jax.nn.sigmoid(rows)
    li = jax.lax.broadcasted_iota(jnp.int32, rows.shape, 2)
    sgm = jnp.where(li >= 4, sg, -1.0)
    mrow = jnp.max(sgm, axis=-1)
    eq = sgm == mrow[..., None]
    idx = jnp.min(jnp.where(eq, li, 127), axis=-1)
    c = (idx - 4).astype(jnp.float32)

    t0 = rows[:, :, 0] * 0.1
    t1 = rows[:, :, 1] * 0.1
    t2 = rows[:, :, 2] * 0.2
    t3 = rows[:, :, 3] * 0.2
    acx = an_ref[0]
    acy = an_ref[1]
    aw = an_ref[2]
    ah = an_ref[3]
    cx = t0 * aw + acx
    cy = t1 * ah + acy
    w = jnp.exp(t2) * aw
    h = jnp.exp(t3) * ah
    x1 = cx - w * 0.5
    y1 = cy - h * 0.5
    x2 = cx + w * 0.5
    y2 = cy + h * 0.5
    area = (x2 - x1) * (y2 - y1)
    conf = s_ref[...]
    s0 = jnp.where(conf > CONF_T, conf, NEG_INF)
    lane = jax.lax.broadcasted_iota(jnp.int32, s0.shape, 1)

    def body(i, s):
        mval = jnp.max(s, axis=1, keepdims=True)
        eq2 = s == mval
        jpos = jnp.min(jnp.where(eq2, lane, NMS_PAD), axis=1, keepdims=True)
        onehot = lane == jpos

        def sel(arr):
            return jnp.sum(jnp.where(onehot, arr, 0.0), axis=1, keepdims=True)

        cxj = sel(cx)
        cyj = sel(cy)
        wj = sel(w)
        hj = sel(h)
        cj = sel(c)
        x1j = cxj - wj * 0.5
        y1j = cyj - hj * 0.5
        x2j = cxj + wj * 0.5
        y2j = cyj + hj * 0.5
        areaj = (x2j - x1j) * (y2j - y1j)
        ix1 = jnp.maximum(x1, x1j)
        iy1 = jnp.maximum(y1, y1j)
        ix2 = jnp.minimum(x2, x2j)
        iy2 = jnp.minimum(y2, y2j)
        inter = jnp.maximum(ix2 - ix1, 0.0) * jnp.maximum(iy2 - iy1, 0.0)
        iou = inter / (area + areaj - inter + 1e-8)
        suppress = ((iou >= IOU_T) & (c == cj)) | onehot
        keep = mval > NEG_INF
        row = jnp.concatenate([cxj, cyj, wj, hj, cj, mval], axis=1)
        row = jnp.where(keep, row, -1.0)
        out_ref[:, pl.ds(i, 1), :] = row[:, None, :]
        return jnp.where(suppress, NEG_INF, s)

    jax.lax.fori_loop(0, MAX_DET, body, s0)


def _nms(rows, an, s):
    return pl.pallas_call(
        _nms_body,
        out_shape=jax.ShapeDtypeStruct((N_IMG, MAX_DET, 6), jnp.float32),
    )(rows, an, s)


def kernel(images, predictions):
    anchors = jnp.asarray(_make_anchors_np(images.shape[1], images.shape[2]))
    m = _confidence(predictions)
    top_s, top_i = jax.lax.top_k(m, PRE_NMS)
    pad = NMS_PAD - PRE_NMS
    top_s = jnp.concatenate(
        [top_s, jnp.full((N_IMG, pad), NEG_INF, jnp.float32)], axis=1)
    top_i = jnp.concatenate(
        [top_i, jnp.zeros((N_IMG, pad), top_i.dtype)], axis=1)
    rows = jnp.take_along_axis(predictions, top_i[..., None], axis=1)
    an = anchors[top_i]
    an = jnp.moveaxis(an, -1, 0)
    return _nms(rows, an, top_s)


# speedup vs baseline: 2.1803x; 2.1803x over previous
import functools

import jax
import jax.numpy as jnp
import numpy as np
from jax import lax
from jax.experimental import pallas as pl
from jax.experimental.pallas import tpu as pltpu
from jax.experimental.pallas import tpu_sc as plsc

IOU_T = 0.5
CONF_T = 0.05
MAX_DET = 100
PRE_NMS = 1000
NMS_PAD = 1024
N_ANCH = 49104
N_ANCH_PAD = 49152
N_IMG = 8
BLK_A = 1024

NEG_INF = float("-inf")


def _make_anchors_np(h, w):
    aspect_ratios = [0.5, 1.0, 2.0]
    scales = [2.0 ** x for x in [0.0, 1.0 / 3.0, 2.0 / 3.0]]
    all_a = []
    for i in range(3, 8):
        area = float((2 ** (i + 2)) ** 2)
        stride = float(2 ** i)
        dims = []
        for r in aspect_ratios:
            ah = np.sqrt(area / r)
            aw = area / ah
            for s in scales:
                dims.append([aw * s, ah * s])
        dims = np.array(dims, np.float32)
        fh = int(np.ceil(h / stride))
        fw = int(np.ceil(w / stride))
        rx = (np.arange(fw, dtype=np.float32) + 0.5) * stride
        ry = (np.arange(fh, dtype=np.float32) + 0.5) * stride
        cx, cy = np.meshgrid(rx, ry)
        centers = np.stack([cx, cy], -1)
        centers = np.tile(centers[:, :, None, :], (1, 1, 9, 1))
        d = np.tile(dims[None, None, :, :], (fh, fw, 1, 1))
        all_a.append(np.concatenate([centers, d], -1).reshape(-1, 4))
    return np.concatenate(all_a, 0)


def _reduce_body(pred_ref, m_ref, c_ref):
    j = pl.program_id(0)
    x = pred_ref[...]
    sg = jax.nn.sigmoid(x)
    li = jax.lax.broadcasted_iota(jnp.int32, x.shape, 2)
    is_cls = li >= 4
    sgm = jnp.where(is_cls, sg, -1.0)
    m = jnp.max(sgm, axis=-1)
    eq = sgm == m[..., None]
    idx = jnp.min(jnp.where(eq, li, 127), axis=-1)
    cls_f = (idx - 4).astype(jnp.float32)
    ga = j * BLK_A + jax.lax.broadcasted_iota(jnp.int32, m.shape, 1)
    valid = ga < N_ANCH
    m_ref[...] = jnp.where(valid, m, NEG_INF)
    c_ref[...] = jnp.where(valid, cls_f, -1.0)


def _reduce_logits(predictions):
    return pl.pallas_call(
        _reduce_body,
        grid=(N_ANCH_PAD // BLK_A,),
        in_specs=[pl.BlockSpec((N_IMG, BLK_A, 84), lambda j: (0, j, 0))],
        out_specs=[
            pl.BlockSpec((N_IMG, BLK_A), lambda j: (0, j)),
            pl.BlockSpec((N_IMG, BLK_A), lambda j: (0, j)),
        ],
        out_shape=[
            jax.ShapeDtypeStruct((N_IMG, N_ANCH_PAD), jnp.float32),
            jax.ShapeDtypeStruct((N_IMG, N_ANCH_PAD), jnp.float32),
        ],
    )(predictions)


_NVREG = N_ANCH_PAD // 16
_U = 8
_SENT = N_ANCH_PAD - 1


def _thresh_body(m_ref, v_ref, ngt_ref):
    u = lax.bitcast_convert_type(m_ref[...], jnp.int32)

    def body(i, carry):
        lo, hi = carry
        mid = lax.shift_right_arithmetic(lo + hi + 1, 1)
        cnt = jnp.sum((u > mid).astype(jnp.int32), axis=1, keepdims=True)
        big = cnt >= PRE_NMS
        return jnp.where(big, mid, lo), jnp.where(big, hi, mid)

    lo0 = jnp.full((N_IMG, 1), -1, jnp.int32)
    hi0 = jnp.full((N_IMG, 1), 0x3F800000, jnp.int32)
    _, hi = lax.fori_loop(0, 31, body, (lo0, hi0))
    v = hi
    n_gt = jnp.sum((u > v).astype(jnp.int32), axis=1, keepdims=True)
    v_ref[...] = jnp.broadcast_to(v, (N_IMG, 128))
    ngt_ref[...] = jnp.broadcast_to(n_gt, (N_IMG, 128))


def _threshold(m):
    return pl.pallas_call(
        _thresh_body,
        out_shape=[
            jax.ShapeDtypeStruct((N_IMG, 128), jnp.int32),
            jax.ShapeDtypeStruct((N_IMG, 128), jnp.int32),
        ],
    )(m)


def _nms_body(bp_ref, an_ref, s_ref, c_ref, out_ref):
    t0 = bp_ref[0] * 0.1
    t1 = bp_ref[1] * 0.1
    t2 = bp_ref[2] * 0.2
    t3 = bp_ref[3] * 0.2
    acx = an_ref[0]
    acy = an_ref[1]
    aw = an_ref[2]
    ah = an_ref[3]
    cx = t0 * aw + acx
    cy = t1 * ah + acy
    w = jnp.exp(t2) * aw
    h = jnp.exp(t3) * ah
    x1 = cx - w * 0.5
    y1 = cy - h * 0.5
    x2 = cx + w * 0.5
    y2 = cy + h * 0.5
    area = (x2 - x1) * (y2 - y1)
    c = c_ref[...]
    conf = s_ref[...]
    s0 = jnp.where(conf > CONF_T, conf, NEG_INF)
    lane = jax.lax.broadcasted_iota(jnp.int32, s0.shape, 1)

    def body(i, s):
        mval = jnp.max(s, axis=1, keepdims=True)
        eq = s == mval
        jpos = jnp.min(jnp.where(eq, lane, NMS_PAD), axis=1, keepdims=True)
        onehot = lane == jpos

        def sel(arr):
            return jnp.sum(jnp.where(onehot, arr, 0.0), axis=1, keepdims=True)

        cxj = sel(cx)
        cyj = sel(cy)
        wj = sel(w)
        hj = sel(h)
        cj = sel(c)
        x1j = cxj - wj * 0.5
        y1j = cyj - hj * 0.5
        x2j = cxj + wj * 0.5
        y2j = cyj + hj * 0.5
        areaj = (x2j - x1j) * (y2j - y1j)
        ix1 = jnp.maximum(x1, x1j)
        iy1 = jnp.maximum(y1, y1j)
        ix2 = jnp.minimum(x2, x2j)
        iy2 = jnp.minimum(y2, y2j)
        inter = jnp.maximum(ix2 - ix1, 0.0) * jnp.maximum(iy2 - iy1, 0.0)
        iou = inter / (area + areaj - inter + 1e-8)
        suppress = ((iou >= IOU_T) & (c == cj)) | onehot
        keep = mval > NEG_INF
        row = jnp.concatenate([cxj, cyj, wj, hj, cj, mval], axis=1)
        row = jnp.where(keep, row, -1.0)
        out_ref[:, pl.ds(i, 1), :] = row[:, None, :]
        return jnp.where(suppress, NEG_INF, s)

    jax.lax.fori_loop(0, MAX_DET, body, s0)


def _nms(bp, an, s, c):
    return pl.pallas_call(
        _nms_body,
        out_shape=jax.ShapeDtypeStruct((N_IMG, MAX_DET, 6), jnp.float32),
    )(bp, an, s, c)


def kernel(images, predictions):
    anchors = jnp.asarray(_make_anchors_np(images.shape[1], images.shape[2]))
    m, cls_f = _reduce_logits(predictions)
    v, n_gt = _threshold(m)
    v1 = v[:, :1]
    k_take = PRE_NMS - n_gt[:, :1]
    u = lax.bitcast_convert_type(m, jnp.int32)
    sel_gt = u > v1
    sel_eq = u == v1
    eqrank = jnp.cumsum(sel_eq.astype(jnp.int32), axis=1)
    selected = sel_gt | (sel_eq & (eqrank <= k_take))
    cs = jnp.cumsum(selected.astype(jnp.int32), axis=1)
    queries = jnp.broadcast_to(
        jnp.arange(1, NMS_PAD + 1, dtype=jnp.int32)[None], (N_IMG, NMS_PAD))
    top_i = jax.vmap(jnp.searchsorted)(cs, queries)
    top_i = jnp.minimum(top_i, _SENT).astype(jnp.int32)
    top_s = jnp.take_along_axis(m, top_i, axis=1)
    top_ic = jnp.minimum(top_i, N_ANCH - 1)
    bp = jnp.take_along_axis(predictions[:, :, :4], top_ic[..., None], axis=1)
    an = anchors[top_ic]
    cg = jnp.take_along_axis(cls_f, top_ic, axis=1)
    bp = jnp.moveaxis(bp, -1, 0)
    an = jnp.moveaxis(an, -1, 0)
    return _nms(bp, an, top_s, cg)


# speedup vs baseline: 2.6930x; 1.2351x over previous
import functools

import jax
import jax.numpy as jnp
import numpy as np
from jax import lax
from jax.experimental import pallas as pl
from jax.experimental.pallas import tpu as pltpu
from jax.experimental.pallas import tpu_sc as plsc

IOU_T = 0.5
CONF_T = 0.05
MAX_DET = 100
PRE_NMS = 1000
NMS_PAD = 1024
N_ANCH = 49104
N_ANCH_PAD = 49152
N_IMG = 8
BLK_A = 1024

NEG_INF = float("-inf")


def _make_anchors_np(h, w):
    aspect_ratios = [0.5, 1.0, 2.0]
    scales = [2.0 ** x for x in [0.0, 1.0 / 3.0, 2.0 / 3.0]]
    all_a = []
    for i in range(3, 8):
        area = float((2 ** (i + 2)) ** 2)
        stride = float(2 ** i)
        dims = []
        for r in aspect_ratios:
            ah = np.sqrt(area / r)
            aw = area / ah
            for s in scales:
                dims.append([aw * s, ah * s])
        dims = np.array(dims, np.float32)
        fh = int(np.ceil(h / stride))
        fw = int(np.ceil(w / stride))
        rx = (np.arange(fw, dtype=np.float32) + 0.5) * stride
        ry = (np.arange(fh, dtype=np.float32) + 0.5) * stride
        cx, cy = np.meshgrid(rx, ry)
        centers = np.stack([cx, cy], -1)
        centers = np.tile(centers[:, :, None, :], (1, 1, 9, 1))
        d = np.tile(dims[None, None, :, :], (fh, fw, 1, 1))
        all_a.append(np.concatenate([centers, d], -1).reshape(-1, 4))
    return np.concatenate(all_a, 0)


def _reduce_body(pred_ref, m_ref, c_ref):
    j = pl.program_id(0)
    x = pred_ref[...]
    sg = jax.nn.sigmoid(x)
    li = jax.lax.broadcasted_iota(jnp.int32, x.shape, 2)
    is_cls = li >= 4
    sgm = jnp.where(is_cls, sg, -1.0)
    m = jnp.max(sgm, axis=-1)
    eq = sgm == m[..., None]
    idx = jnp.min(jnp.where(eq, li, 127), axis=-1)
    cls_f = (idx - 4).astype(jnp.float32)
    ga = j * BLK_A + jax.lax.broadcasted_iota(jnp.int32, m.shape, 1)
    valid = ga < N_ANCH
    m_ref[...] = jnp.where(valid, m, NEG_INF)
    c_ref[...] = jnp.where(valid, cls_f, -1.0)


def _reduce_logits(predictions):
    return pl.pallas_call(
        _reduce_body,
        grid=(N_ANCH_PAD // BLK_A,),
        in_specs=[pl.BlockSpec((N_IMG, BLK_A, 84), lambda j: (0, j, 0))],
        out_specs=[
            pl.BlockSpec((N_IMG, BLK_A), lambda j: (0, j)),
            pl.BlockSpec((N_IMG, BLK_A), lambda j: (0, j)),
        ],
        out_shape=[
            jax.ShapeDtypeStruct((N_IMG, N_ANCH_PAD), jnp.float32),
            jax.ShapeDtypeStruct((N_IMG, N_ANCH_PAD), jnp.float32),
        ],
    )(predictions)


_NVREG = N_ANCH_PAD // 16
_U = 8
_SENT = N_ANCH_PAD - 1


def _thresh_body(m_ref, cs_ref):
    u = lax.bitcast_convert_type(m_ref[...], jnp.int32)

    def body(i, carry):
        lo, hi = carry
        mid = lax.shift_right_arithmetic(lo + hi + 1, 1)
        cnt = jnp.sum((u > mid).astype(jnp.int32), axis=(1, 2), keepdims=True)
        big = cnt >= PRE_NMS
        return jnp.where(big, mid, lo), jnp.where(big, hi, mid)

    lo0 = jnp.full((N_IMG, 1, 1), -1, jnp.int32)
    hi0 = jnp.full((N_IMG, 1, 1), 0x3F800000, jnp.int32)
    _, v = lax.fori_loop(0, 31, body, (lo0, hi0))
    n_gt = jnp.sum((u > v).astype(jnp.int32), axis=(1, 2), keepdims=True)
    k_take = PRE_NMS - n_gt

    li = lax.broadcasted_iota(jnp.int32, (128, 128), 0)
    mi = lax.broadcasted_iota(jnp.int32, (128, 128), 1)
    tri_incl = (li <= mi).astype(jnp.float32)
    ci = lax.broadcasted_iota(jnp.int32, (384, 384), 0)
    di = lax.broadcasted_iota(jnp.int32, (384, 384), 1)
    tri_excl = (ci < di).astype(jnp.float32)

    def hier_cumsum(mask_f32):
        within = jax.lax.dot_general(
            mask_f32, tri_incl, (((2,), (0,)), ((), ())),
            preferred_element_type=jnp.float32)
        sums = within[:, :, 127]
        prefix = jax.lax.dot_general(
            sums, tri_excl, (((1,), (0,)), ((), ())),
            preferred_element_type=jnp.float32)
        return within + prefix[:, :, None]

    sel_eq = u == v
    eqrank = hier_cumsum(sel_eq.astype(jnp.float32))
    selected = (u > v) | (sel_eq & (eqrank <= k_take.astype(jnp.float32)))
    cs = hier_cumsum(selected.astype(jnp.float32))
    cs_ref[...] = cs.astype(jnp.int32)


def _threshold(m):
    m3 = m.reshape(N_IMG, N_ANCH_PAD // 128, 128)
    cs = pl.pallas_call(
        _thresh_body,
        out_shape=jax.ShapeDtypeStruct((N_IMG, N_ANCH_PAD // 128, 128),
                                       jnp.int32),
    )(m3)
    return cs.reshape(N_IMG, N_ANCH_PAD)


def _nms_body(bp_ref, an_ref, s_ref, c_ref, out_ref):
    t0 = bp_ref[0] * 0.1
    t1 = bp_ref[1] * 0.1
    t2 = bp_ref[2] * 0.2
    t3 = bp_ref[3] * 0.2
    acx = an_ref[0]
    acy = an_ref[1]
    aw = an_ref[2]
    ah = an_ref[3]
    cx = t0 * aw + acx
    cy = t1 * ah + acy
    w = jnp.exp(t2) * aw
    h = jnp.exp(t3) * ah
    x1 = cx - w * 0.5
    y1 = cy - h * 0.5
    x2 = cx + w * 0.5
    y2 = cy + h * 0.5
    area = (x2 - x1) * (y2 - y1)
    c = c_ref[...]
    conf = s_ref[...]
    s0 = jnp.where(conf > CONF_T, conf, NEG_INF)
    lane = jax.lax.broadcasted_iota(jnp.int32, s0.shape, 1)

    def body(i, s):
        mval = jnp.max(s, axis=1, keepdims=True)
        eq = s == mval
        jpos = jnp.min(jnp.where(eq, lane, NMS_PAD), axis=1, keepdims=True)
        onehot = lane == jpos

        def sel(arr):
            return jnp.sum(jnp.where(onehot, arr, 0.0), axis=1, keepdims=True)

        cxj = sel(cx)
        cyj = sel(cy)
        wj = sel(w)
        hj = sel(h)
        cj = sel(c)
        x1j = cxj - wj * 0.5
        y1j = cyj - hj * 0.5
        x2j = cxj + wj * 0.5
        y2j = cyj + hj * 0.5
        areaj = (x2j - x1j) * (y2j - y1j)
        ix1 = jnp.maximum(x1, x1j)
        iy1 = jnp.maximum(y1, y1j)
        ix2 = jnp.minimum(x2, x2j)
        iy2 = jnp.minimum(y2, y2j)
        inter = jnp.maximum(ix2 - ix1, 0.0) * jnp.maximum(iy2 - iy1, 0.0)
        iou = inter / (area + areaj - inter + 1e-8)
        suppress = ((iou >= IOU_T) & (c == cj)) | onehot
        keep = mval > NEG_INF
        row = jnp.concatenate([cxj, cyj, wj, hj, cj, mval], axis=1)
        row = jnp.where(keep, row, -1.0)
        out_ref[:, pl.ds(i, 1), :] = row[:, None, :]
        return jnp.where(suppress, NEG_INF, s)

    jax.lax.fori_loop(0, MAX_DET, body, s0)


def _nms(bp, an, s, c):
    return pl.pallas_call(
        _nms_body,
        out_shape=jax.ShapeDtypeStruct((N_IMG, MAX_DET, 6), jnp.float32),
    )(bp, an, s, c)


def kernel(images, predictions):
    anchors = jnp.asarray(_make_anchors_np(images.shape[1], images.shape[2]))
    m, cls_f = _reduce_logits(predictions)
    cs = _threshold(m)
    queries = jnp.broadcast_to(
        jnp.arange(1, NMS_PAD + 1, dtype=jnp.int32)[None], (N_IMG, NMS_PAD))
    top_i = jax.vmap(jnp.searchsorted)(cs, queries)
    top_i = jnp.minimum(top_i, _SENT).astype(jnp.int32)
    top_s = jnp.take_along_axis(m, top_i, axis=1)
    top_ic = jnp.minimum(top_i, N_ANCH - 1)
    bp = jnp.take_along_axis(predictions[:, :, :4], top_ic[..., None], axis=1)
    an = anchors[top_ic]
    cg = jnp.take_along_axis(cls_f, top_ic, axis=1)
    bp = jnp.moveaxis(bp, -1, 0)
    an = jnp.moveaxis(an, -1, 0)
    return _nms(bp, an, top_s, cg)
